# baseline (speedup 1.0000x reference)
import functools

import jax
import jax.numpy as jnp
import numpy as np
from jax import lax
from jax.experimental import pallas as pl
from jax.experimental.pallas import tpu as pltpu
from jax.experimental.pallas import tpu_sc as plsc

BATCH = 1024
SEQ = 200
DIM = 300
TOTAL = BATCH * SEQ
M = 120
NPIECE = 3
PAIR = 2 * DIM

CHUNK = 80
NPAIR = CHUNK // 2

GROUPS = tuple(range(0, PAIR - 16, 16)) + (PAIR - 16,)


def _pattern_table():
    tab = np.zeros((4 * len(GROUPS), 16), np.int32)
    lane = np.arange(16)
    for co in range(4):
        s0 = 60 * (co >> 1)
        s1 = 60 * (co & 1)
        for ci, c in enumerate(GROUPS):
            w = lane + c
            second = (w >= DIM).astype(np.int32)
            off = w - DIM * second
            t = off + np.where(second == 1, s1, s0)
            j = t // M
            col = t % M
            tab[co * len(GROUPS) + ci] = (j * CHUNK + second) * 1024 + col
    return tab


@functools.lru_cache(maxsize=None)
def _build(total):
    info = plsc.get_sparse_core_info()
    nw = info.num_cores * info.num_subcores
    b_per_w = total // nw
    n_chunks = b_per_w // CHUNK
    assert n_chunks % 2 == 0
    ngroups = len(GROUPS)
    mesh = plsc.VectorSubcoreMesh(core_axis_name="c", subcore_axis_name="s")

    @functools.partial(
        pl.kernel,
        mesh=mesh,
        compiler_params=pltpu.CompilerParams(
            use_tc_tiling_on_sc=False, needs_layout_passes=False
        ),
        out_type=jax.ShapeDtypeStruct((total * DIM // PAIR, PAIR), jnp.float32),
        scratch_types=[
            pltpu.VMEM((b_per_w,), jnp.int32),
            pltpu.VMEM((4 * ngroups, 16), jnp.int32),
            pltpu.VMEM((NPIECE, CHUNK), jnp.int32),
            pltpu.VMEM((NPIECE, CHUNK), jnp.int32),
            pltpu.VMEM((NPIECE * CHUNK, M), jnp.float32),
            pltpu.VMEM((NPIECE * CHUNK, M), jnp.float32),
            pltpu.VMEM((NPAIR, PAIR), jnp.float32),
            pltpu.VMEM((NPAIR, PAIR), jnp.float32),
            pltpu.SemaphoreType.DMA,
            pltpu.SemaphoreType.DMA,
            pltpu.SemaphoreType.DMA,
            pltpu.SemaphoreType.DMA,
        ],
    )
    def gather_kernel(idx_hbm, table_hbm, ptab_hbm, out_hbm, idx_all, ptab,
                      pidx0, pidx1, rows0, rows1, flat0, flat1,
                      gsem0, gsem1, osem0, osem1):
        pidx = (pidx0, pidx1)
        rows = (rows0, rows1)
        flats = (flat0, flat1)
        gsems = (gsem0, gsem1)
        osems = (osem0, osem1)
        wid = lax.axis_index("s") * info.num_cores + lax.axis_index("c")
        wbase = wid * b_per_w
        qw = wbase // 2

        pltpu.sync_copy(idx_hbm.at[pl.ds(wbase, b_per_w)], idx_all)
        pltpu.sync_copy(ptab_hbm, ptab)

        def build_piece_ids(i, b):
            def body(v, carry):
                g = idx_all[pl.ds(i * CHUNK + v * 16, 16)]
                p0 = lax.shift_right_logical(g * 5, 1)
                pidx[b][0, pl.ds(v * 16, 16)] = p0
                pidx[b][1, pl.ds(v * 16, 16)] = p0 + 1
                pidx[b][2, pl.ds(v * 16, 16)] = p0 + 2
                return carry

            lax.fori_loop(0, CHUNK // 16, body, 0)

        def start_gathers(i, b):
            build_piece_ids(i, b)
            for j in range(NPIECE):
                pltpu.async_copy(
                    table_hbm.at[pidx[b].at[j]],
                    rows[b].at[pl.ds(j * CHUNK, CHUNK)],
                    gsems[b])

        def wait_gathers(b):
            pltpu.make_async_copy(
                table_hbm.at[pl.ds(0, NPIECE * CHUNK)], rows[b], gsems[b]
            ).wait()

        def out_dst(i):
            return out_hbm.at[pl.ds(qw + i * NPAIR, NPAIR)]

        def repack(i, b):
            def pair_body(q, carry):
                v = idx_all[pl.ds(i * CHUNK + 2 * q, 16)]
                co = (v[0] & 1) * (2 * ngroups) + (v[1] & 1) * ngroups
                r0 = 2 * q
                for ci, c in enumerate(GROUPS):
                    p = ptab[co + ci, pl.ds(0, 16)]
                    src_row = lax.shift_right_logical(p, 10) + r0
                    src_col = p & 1023
                    vals = plsc.load_gather(rows[b], [src_row, src_col])
                    flats[b][q, pl.ds(c, 16)] = vals
                return carry

            lax.fori_loop(0, NPAIR, pair_body, 0)

        start_gathers(0, 0)
        start_gathers(1, 1)

        def outer(g, carry):
            for b in range(2):
                i = 2 * g + b
                wait_gathers(b)

                @pl.when(i >= 2)
                def _():
                    pltpu.make_async_copy(flats[b], out_dst(i - 2), osems[b]).wait()

                repack(i, b)

                @pl.when(i + 2 < n_chunks)
                def _():
                    start_gathers(i + 2, b)

                pltpu.async_copy(flats[b], out_dst(i), osems[b])
            return carry

        lax.fori_loop(0, n_chunks // 2, outer, 0)

        pltpu.make_async_copy(flats[0], out_dst(n_chunks - 2), osems[0]).wait()
        pltpu.make_async_copy(flats[1], out_dst(n_chunks - 1), osems[1]).wait()

    return gather_kernel


def kernel(x, word_vectors):
    idx = x.reshape(-1).astype(jnp.int32)
    table_p = word_vectors.reshape(-1, M)
    ptab = jnp.asarray(_pattern_table())
    out = _build(TOTAL)(idx, table_p, ptab)
    return out.reshape(BATCH, SEQ, DIM)


# speedup vs baseline: 4.4227x; 4.4227x over previous
import functools

import jax
import jax.numpy as jnp
from jax import lax
from jax.experimental import pallas as pl
from jax.experimental.pallas import tpu as pltpu
from jax.experimental.pallas import tpu_sc as plsc

BATCH = 1024
SEQ = 200
DIM = 300
TOTAL = BATCH * SEQ
LANES = 128
TAIL = DIM - 2 * LANES

CHUNK = 40
NGROUP = CHUNK // 8
NBUF = 4


@functools.lru_cache(maxsize=None)
def _build(total):
    info = plsc.get_sparse_core_info()
    nw = info.num_cores * info.num_subcores
    b_per_w = total // nw
    n_chunks = b_per_w // CHUNK
    assert n_chunks % NBUF == 0
    mesh = plsc.VectorSubcoreMesh(core_axis_name="c", subcore_axis_name="s")

    @functools.partial(
        pl.kernel,
        mesh=mesh,
        out_type=jax.ShapeDtypeStruct((total // 8, 8, 3 * LANES), jnp.float32),
        scratch_types=[
            pltpu.VMEM((b_per_w,), jnp.int32),
            *[pltpu.VMEM((CHUNK, LANES), jnp.float32) for _ in range(3 * NBUF)],
            *[pltpu.SemaphoreType.DMA for _ in range(2 * NBUF)],
        ],
    )
    def gather_kernel(idx_hbm, table_hbm, aux_hbm, out_hbm, idx_all,
                      *bufs_sems):
        bufs = tuple(tuple(bufs_sems[3 * b:3 * b + 3]) for b in range(NBUF))
        gsems = bufs_sems[3 * NBUF:4 * NBUF]
        osems = bufs_sems[4 * NBUF:]
        wid = lax.axis_index("s") * info.num_cores + lax.axis_index("c")
        wbase = wid * b_per_w
        wg = wbase // 8

        pltpu.sync_copy(idx_hbm.at[pl.ds(wbase, b_per_w)], idx_all)

        def start_gathers(i, b):
            sl = idx_all.at[pl.ds(i * CHUNK, CHUNK)]
            m0, m1, m2 = bufs[b]
            pltpu.async_copy(
                table_hbm.at[plsc.Indices(sl), pl.ds(0, LANES)], m0, gsems[b])
            pltpu.async_copy(
                table_hbm.at[plsc.Indices(sl), pl.ds(LANES, LANES)], m1,
                gsems[b])
            pltpu.async_copy(aux_hbm.at[plsc.Indices(sl)], m2, gsems[b])

        def wait_gathers(b):
            for m in bufs[b]:
                pltpu.make_async_copy(
                    table_hbm.at[pl.ds(0, CHUNK), pl.ds(0, LANES)], m, gsems[b]
                ).wait()

        def issue_tile_writes(i, b):
            m0, m1, m2 = bufs[b]

            def group_body(g, carry):
                dst = out_hbm.at[wg + i * NGROUP + g]
                rs = pl.ds(8 * g, 8)
                pltpu.async_copy(m0.at[rs], dst.at[:, pl.ds(0, LANES)],
                                 osems[b])
                pltpu.async_copy(m1.at[rs], dst.at[:, pl.ds(LANES, LANES)],
                                 osems[b])
                pltpu.async_copy(m2.at[rs], dst.at[:, pl.ds(2 * LANES, LANES)],
                                 osems[b])
                return carry

            lax.fori_loop(0, NGROUP, group_body, 0)

        def wait_tile_writes(b):
            for m in bufs[b]:
                pltpu.make_async_copy(
                    table_hbm.at[pl.ds(0, CHUNK), pl.ds(0, LANES)], m, osems[b]
                ).wait()

        for j in range(NBUF - 1):
            start_gathers(j, j)

        def outer(g, carry):
            for b in range(NBUF):
                i = NBUF * g + b
                wait_gathers(b)
                issue_tile_writes(i, b)
                nb = (b + NBUF - 1) % NBUF

                @pl.when(i + NBUF - 1 < n_chunks)
                def _():
                    @pl.when(i >= 1)
                    def _():
                        wait_tile_writes(nb)

                    start_gathers(i + NBUF - 1, nb)
            return carry

        lax.fori_loop(0, n_chunks // NBUF, outer, 0)

        for j in range(NBUF):
            wait_tile_writes((n_chunks - NBUF + j) % NBUF)

    return gather_kernel


def kernel(x, word_vectors):
    idx = x.reshape(-1).astype(jnp.int32)
    aux = jnp.pad(word_vectors[:, 2 * LANES:], ((0, 0), (0, LANES - TAIL)))
    out = _build(TOTAL)(idx, word_vectors, aux)
    return out[:, :, :DIM].reshape(BATCH, SEQ, DIM)
